# baseline (speedup 1.0000x reference)
import functools

import jax
import jax.numpy as jnp
from jax import lax
from jax.experimental import pallas as pl
from jax.experimental.pallas import tpu as pltpu
from jax.experimental.pallas import tpu_sc as plsc

_N = 262144
_NB = 8
_NSUB = 16
_CHUNK = _N // _NSUB
_NV = _CHUNK // 16
_B = 2048
_BV = _B // 16
_U = 4
_NVU = _NV // _U
_SUB = _CHUNK // _U
_IDX_BITS = 18
_IDX_MASK = (1 << _IDX_BITS) - 1

_SHIFTS = (
    (2, 1, 0),
    (1, 2, 0),
    (1, 0, 2),
    (2, 0, 1),
)


def _spread10(v):
    v = (v | (v << 16)) & 0x030000FF
    v = (v | (v << 8)) & 0x0300F00F
    v = (v | (v << 4)) & 0x030C30C3
    v = (v | (v << 2)) & 0x09249249
    return v


def _sel(cid, a, b):
    return jnp.where(cid == 0, jnp.int32(a), jnp.int32(b))


def _sort_body(xs, ys, zs, bs, sorted_out, inv_out, boff_out,
               mk, pv, pos, hists, bases, hist, gl, sbuf, grid, sem):
    cid = lax.axis_index("c")
    sid = lax.axis_index("s")
    slc = pl.ds(sid * _CHUNK, _CHUNK)
    zeros16 = jnp.zeros((16,), jnp.int32)
    ii16 = lax.iota(jnp.int32, 16)

    def loop(n, fn):
        def body(i, _):
            fn(i)
            return 0
        lax.fori_loop(jnp.int32(0), jnp.int32(n), body, 0)

    def sub_ds(k, i):
        return pl.ds(k * _SUB + i * 16, 16)

    def zero_hists():
        def z(j):
            hists[pl.ds(j * 16, 16)] = zeros16
        loop(_U * _BV, z)

    def exchange_and_scan(p, r):
        def merge(j):
            v = pl.ds(j * 16, 16)
            h = hists[v]
            for k in range(1, _U):
                h = h + hists[pl.ds(k * _B + j * 16, 16)]
            hist[v] = h
        loop(_BV, merge)
        pltpu.sync_copy(hist, grid.at[pl.ds(sid * _B, _B)])
        plsc.subcore_barrier()
        pltpu.sync_copy(grid, gl)

        def scan_body(j, carry):
            v = pl.ds(j * 16, 16)
            acc = zeros16
            pre = zeros16
            for t in range(_NSUB):
                rowv = gl[pl.ds(t * _B + j * 16, 16)]
                pre = pre + jnp.where(t < sid, rowv, zeros16)
                acc = acc + rowv
            excl = plsc.cumsum(acc) - acc
            bases[v] = pre + excl + carry
            return carry + jnp.sum(acc, dtype=jnp.int32)
        lax.fori_loop(jnp.int32(0), jnp.int32(_BV), scan_body, jnp.int32(0))

        if r == 0 and p == 2:
            @pl.when(jnp.logical_and(cid == 0, sid == 0))
            def _():
                bsel = jnp.minimum(ii16 * (_B // _NB), _B - 1)
                g = plsc.load_gather(bases, [bsel])
                hist[pl.ds(0, 16)] = jnp.where(ii16 < _NB, g, jnp.int32(_N))
                pltpu.sync_copy(hist.at[pl.ds(0, 16)], boff_out)

        def mkbases(j):
            v = pl.ds(j * 16, 16)
            b = bases[v]
            for k in range(1, _U):
                b = b + hists[pl.ds((k - 1) * _B + j * 16, 16)]
                bases[pl.ds(k * _B + j * 16, 16)] = b
        loop(_BV, mkbases)

    def hist_one(k, d):
        cnt, last = plsc.scan_count(d)
        plsc.addupdate_scatter(hists, [d + (k * _B)], cnt, mask=last)

    def rank_one(k, i, d):
        dk = d + (k * _B)
        off = plsc.load_gather(bases, [dk])
        cnt, last = plsc.scan_count(dk)
        pos[sub_ds(k, i)] = off + cnt - 1
        plsc.addupdate_scatter(bases, [dk], cnt, mask=last)

    for r in range(2):
        row = cid * 2 + r
        shx = _sel(cid, _SHIFTS[r][0], _SHIFTS[2 + r][0])
        shy = _sel(cid, _SHIFTS[r][1], _SHIFTS[2 + r][1])
        shz = _sel(cid, _SHIFTS[r][2], _SHIFTS[2 + r][2])

        pltpu.sync_copy(xs.at[slc], pos)
        pltpu.sync_copy(ys.at[slc], gl.at[pl.ds(0, _CHUNK)])
        pltpu.sync_copy(zs.at[slc], gl.at[pl.ds(_CHUNK, _CHUNK)])
        pltpu.sync_copy(bs.at[slc], pv)
        with jax.named_scope("zero0"):
            zero_hists()

        base_idx = sid * _CHUNK
        def encode(i):
            for k in range(_U):
                ds = sub_ds(k, i)
                x = pos[ds]
                y = gl[pl.ds(k * _SUB + i * 16, 16)]
                z = gl[pl.ds(_CHUNK + k * _SUB + i * 16, 16)]
                b = pv[ds]
                code = ((_spread10(x) << shx) | (_spread10(y) << shy)
                        | (_spread10(z) << shz))
                mk[ds] = code
                pv[ds] = (base_idx + k * _SUB + i * 16 + ii16) | (b << _IDX_BITS)
                hist_one(k, code & (_B - 1))
        with jax.named_scope("encode"):
            loop(_NVU, encode)

        for p in range(3):
            if p == 0:
                digit = lambda k: k & (_B - 1)
            elif p == 1:
                digit = lambda k: (k >> 11) & (_B - 1)
            else:
                digit = lambda k: (k >> 21) | (((k >> _IDX_BITS) & 7) << 8)

            if p > 0:
                with jax.named_scope("sweep1"):
                    zero_hists()
                    def sweep1(i):
                        for k in range(_U):
                            hist_one(k, digit(mk[sub_ds(k, i)]))
                    loop(_NVU, sweep1)

            with jax.named_scope("exscan"):
                exchange_and_scan(p, r)

            if p == 1:
                def sweep2(i):
                    for k in range(_U):
                        ds = sub_ds(k, i)
                        m = mk[ds]
                        rank_one(k, i, digit(m))
                        pv[ds] = pv[ds] | ((m >> 22) << 21)
            elif p == 2:
                def sweep2(i):
                    for k in range(_U):
                        ds = sub_ds(k, i)
                        m = mk[ds]
                        rank_one(k, i, digit(m))
                        pv[ds] = m & _IDX_MASK
            else:
                def sweep2(i):
                    for k in range(_U):
                        rank_one(k, i, digit(mk[sub_ds(k, i)]))
            with jax.named_scope("sweep2"):
                loop(_NVU, sweep2)

            if p == 0:
              with jax.named_scope("scat0"):
                  pltpu.async_copy(mk, sbuf.at[pos], sem).wait()
                  plsc.subcore_barrier()
                  pltpu.sync_copy(sbuf.at[slc], mk)
                  plsc.subcore_barrier()
                  pltpu.async_copy(pv, sbuf.at[pos], sem).wait()
                  plsc.subcore_barrier()
                  pltpu.sync_copy(sbuf.at[slc], pv)
            elif p == 1:
              with jax.named_scope("scat1"):
                  pltpu.async_copy(pv, sbuf.at[pos], sem).wait()
                  plsc.subcore_barrier()
                  pltpu.sync_copy(sbuf.at[slc], mk)
            else:
              with jax.named_scope("scat2"):
                  pltpu.async_copy(pv, sbuf.at[pos], sem).wait()
                  plsc.subcore_barrier()
                  pltpu.sync_copy(sbuf.at[slc], sorted_out.at[row, slc])
                  plsc.subcore_barrier()
                  pltpu.async_copy(pos, sbuf.at[pv], sem).wait()
                  plsc.subcore_barrier()
                  pltpu.sync_copy(sbuf.at[slc], inv_out.at[row, slc])
                  plsc.subcore_barrier()


@functools.cache
def _build_sort_kernel():
    mesh = plsc.VectorSubcoreMesh(core_axis_name="c", subcore_axis_name="s")
    return pl.kernel(
        _sort_body,
        out_type=(
            jax.ShapeDtypeStruct((4, _N), jnp.int32),
            jax.ShapeDtypeStruct((4, _N), jnp.int32),
            jax.ShapeDtypeStruct((16,), jnp.int32),
        ),
        mesh=mesh,
        scratch_types=[
            pltpu.VMEM((_CHUNK,), jnp.int32),
            pltpu.VMEM((_CHUNK,), jnp.int32),
            pltpu.VMEM((_CHUNK,), jnp.int32),
            pltpu.VMEM((_U * _B,), jnp.int32),
            pltpu.VMEM((_U * _B,), jnp.int32),
            pltpu.VMEM((_B,), jnp.int32),
            pltpu.VMEM((_NSUB * _B,), jnp.int32),
            pltpu.VMEM_SHARED((_N,), jnp.int32),
            pltpu.VMEM_SHARED((_NSUB * _B,), jnp.int32),
            pltpu.SemaphoreType.DMA,
        ],
        compiler_params=pltpu.CompilerParams(needs_layout_passes=False),
    )


def kernel(features, batch_ids, coords):
    coords32 = coords.astype(jnp.int32)
    xs = coords32[:, 0]
    ys = coords32[:, 1]
    zs = coords32[:, 2]
    bs = batch_ids.astype(jnp.int32)
    sorted32, inv32, boff16 = _build_sort_kernel()(xs, ys, zs, bs)
    sorted_idxs = sorted32.astype(jnp.int64)
    inv_sorted_idxs = inv32.astype(jnp.int64)
    batch_offsets = boff16[:9]
    max_counts = jnp.max(boff16[1:9] - boff16[:8]).astype(jnp.int64)
    return (features, sorted_idxs, inv_sorted_idxs, batch_offsets, max_counts)


# speedup vs baseline: 1.0045x; 1.0045x over previous
import functools

import jax
import jax.numpy as jnp
from jax import lax
from jax.experimental import pallas as pl
from jax.experimental.pallas import tpu as pltpu
from jax.experimental.pallas import tpu_sc as plsc

_N = 262144
_NB = 8
_NSUB = 16
_CHUNK = _N // _NSUB
_NV = _CHUNK // 16
_B = 2048
_BV = _B // 16
_U = 4
_NVU = _NV // _U
_SUB = _CHUNK // _U
_IDX_BITS = 18
_IDX_MASK = (1 << _IDX_BITS) - 1

_SHIFTS = (
    (2, 1, 0),
    (1, 2, 0),
    (1, 0, 2),
    (2, 0, 1),
)


def _spread10(v):
    v = (v | (v << 16)) & 0x030000FF
    v = (v | (v << 8)) & 0x0300F00F
    v = (v | (v << 4)) & 0x030C30C3
    v = (v | (v << 2)) & 0x09249249
    return v


def _sel(cid, a, b):
    return jnp.where(cid == 0, jnp.int32(a), jnp.int32(b))


def _sort_body(xs, ys, zs, bs, sorted_out, inv_out, boff_out,
               mk, pv, pos, hists, bases, hist, gl, sbuf, grid, sem):
    cid = lax.axis_index("c")
    sid = lax.axis_index("s")
    slc = pl.ds(sid * _CHUNK, _CHUNK)
    zeros16 = jnp.zeros((16,), jnp.int32)
    ii16 = lax.iota(jnp.int32, 16)

    def loop(n, fn):
        def body(i, _):
            fn(i)
            return 0
        lax.fori_loop(jnp.int32(0), jnp.int32(n), body, 0)

    def sub_ds(k, i):
        return pl.ds(k * _SUB + i * 16, 16)

    def zero_hists():
        def z(j):
            hists[pl.ds(j * 16, 16)] = zeros16
        loop(_U * _BV, z)

    def exchange_and_scan(p, r):
        def merge(j):
            v = pl.ds(j * 16, 16)
            h = hists[v]
            for k in range(1, _U):
                h = h + hists[pl.ds(k * _B + j * 16, 16)]
            hist[v] = h
        loop(_BV, merge)
        pltpu.sync_copy(hist, grid.at[pl.ds(sid * _B, _B)])
        plsc.subcore_barrier()
        pltpu.sync_copy(grid, gl)

        def scan_body(j, carry):
            v = pl.ds(j * 16, 16)
            acc = zeros16
            pre = zeros16
            for t in range(_NSUB):
                rowv = gl[pl.ds(t * _B + j * 16, 16)]
                pre = pre + jnp.where(t < sid, rowv, zeros16)
                acc = acc + rowv
            excl = plsc.cumsum(acc) - acc
            bases[v] = pre + excl + carry
            return carry + jnp.sum(acc, dtype=jnp.int32)
        lax.fori_loop(jnp.int32(0), jnp.int32(_BV), scan_body, jnp.int32(0))

        if r == 0 and p == 2:
            @pl.when(jnp.logical_and(cid == 0, sid == 0))
            def _():
                bsel = jnp.minimum(ii16 * (_B // _NB), _B - 1)
                g = plsc.load_gather(bases, [bsel])
                hist[pl.ds(0, 16)] = jnp.where(ii16 < _NB, g, jnp.int32(_N))
                pltpu.sync_copy(hist.at[pl.ds(0, 16)], boff_out)

        def mkbases(j):
            v = pl.ds(j * 16, 16)
            b = bases[v]
            for k in range(1, _U):
                b = b + hists[pl.ds((k - 1) * _B + j * 16, 16)]
                bases[pl.ds(k * _B + j * 16, 16)] = b
        loop(_BV, mkbases)

    def hist_one(k, d):
        cnt, last = plsc.scan_count(d)
        plsc.addupdate_scatter(hists, [d + (k * _B)], cnt, mask=last)

    def rank_one(k, i, d):
        dk = d + (k * _B)
        off = plsc.load_gather(bases, [dk])
        cnt, last = plsc.scan_count(dk)
        pos[sub_ds(k, i)] = off + cnt - 1
        plsc.addupdate_scatter(bases, [dk], cnt, mask=last)

    for r in range(2):
        row = cid * 2 + r
        shx = _sel(cid, _SHIFTS[r][0], _SHIFTS[2 + r][0])
        shy = _sel(cid, _SHIFTS[r][1], _SHIFTS[2 + r][1])
        shz = _sel(cid, _SHIFTS[r][2], _SHIFTS[2 + r][2])

        pltpu.sync_copy(xs.at[slc], pos)
        pltpu.sync_copy(ys.at[slc], gl.at[pl.ds(0, _CHUNK)])
        pltpu.sync_copy(zs.at[slc], gl.at[pl.ds(_CHUNK, _CHUNK)])
        pltpu.sync_copy(bs.at[slc], pv)
        zero_hists()

        base_idx = sid * _CHUNK
        def encode(i):
            for k in range(_U):
                ds = sub_ds(k, i)
                x = pos[ds]
                y = gl[pl.ds(k * _SUB + i * 16, 16)]
                z = gl[pl.ds(_CHUNK + k * _SUB + i * 16, 16)]
                b = pv[ds]
                code = ((_spread10(x) << shx) | (_spread10(y) << shy)
                        | (_spread10(z) << shz))
                mk[ds] = code
                pv[ds] = (base_idx + k * _SUB + i * 16 + ii16) | (b << _IDX_BITS)
                hist_one(k, code & (_B - 1))
        loop(_NVU, encode)

        for p in range(3):
            if p == 0:
                digit = lambda k: k & (_B - 1)
            elif p == 1:
                digit = lambda k: (k >> 11) & (_B - 1)
            else:
                digit = lambda k: (k >> 21) | (((k >> _IDX_BITS) & 7) << 8)

            if p > 0:
                zero_hists()
                def sweep1(i):
                    for k in range(_U):
                        hist_one(k, digit(mk[sub_ds(k, i)]))
                loop(_NVU, sweep1)

            exchange_and_scan(p, r)

            if p == 1:
                def sweep2(i):
                    for k in range(_U):
                        ds = sub_ds(k, i)
                        m = mk[ds]
                        rank_one(k, i, digit(m))
                        pv[ds] = pv[ds] | ((m >> 22) << 21)
            elif p == 2:
                def sweep2(i):
                    for k in range(_U):
                        ds = sub_ds(k, i)
                        m = mk[ds]
                        rank_one(k, i, digit(m))
                        pv[ds] = m & _IDX_MASK
            else:
                def sweep2(i):
                    for k in range(_U):
                        rank_one(k, i, digit(mk[sub_ds(k, i)]))
            loop(_NVU, sweep2)

            if p == 0:
                pltpu.async_copy(mk, sbuf.at[pos], sem).wait()
                plsc.subcore_barrier()
                pltpu.sync_copy(sbuf.at[slc], mk)
                plsc.subcore_barrier()
                pltpu.async_copy(pv, sbuf.at[pos], sem).wait()
                plsc.subcore_barrier()
                pltpu.sync_copy(sbuf.at[slc], pv)
            elif p == 1:
                pltpu.async_copy(pv, sbuf.at[pos], sem).wait()
                plsc.subcore_barrier()
                pltpu.sync_copy(sbuf.at[slc], mk)
            else:
                pltpu.async_copy(pv, sbuf.at[pos], sem).wait()
                plsc.subcore_barrier()
                pltpu.sync_copy(sbuf.at[slc], sorted_out.at[row, slc])
                plsc.subcore_barrier()
                pltpu.async_copy(pos, sbuf.at[pv], sem).wait()
                plsc.subcore_barrier()
                pltpu.sync_copy(sbuf.at[slc], inv_out.at[row, slc])
                plsc.subcore_barrier()


@functools.cache
def _build_sort_kernel():
    mesh = plsc.VectorSubcoreMesh(core_axis_name="c", subcore_axis_name="s")
    return pl.kernel(
        _sort_body,
        out_type=(
            jax.ShapeDtypeStruct((4, _N), jnp.int32),
            jax.ShapeDtypeStruct((4, _N), jnp.int32),
            jax.ShapeDtypeStruct((16,), jnp.int32),
        ),
        mesh=mesh,
        scratch_types=[
            pltpu.VMEM((_CHUNK,), jnp.int32),
            pltpu.VMEM((_CHUNK,), jnp.int32),
            pltpu.VMEM((_CHUNK,), jnp.int32),
            pltpu.VMEM((_U * _B,), jnp.int32),
            pltpu.VMEM((_U * _B,), jnp.int32),
            pltpu.VMEM((_B,), jnp.int32),
            pltpu.VMEM((_NSUB * _B,), jnp.int32),
            pltpu.VMEM_SHARED((_N,), jnp.int32),
            pltpu.VMEM_SHARED((_NSUB * _B,), jnp.int32),
            pltpu.SemaphoreType.DMA,
        ],
        compiler_params=pltpu.CompilerParams(needs_layout_passes=False),
    )


def kernel(features, batch_ids, coords):
    coords32 = coords.astype(jnp.int32)
    xs = coords32[:, 0]
    ys = coords32[:, 1]
    zs = coords32[:, 2]
    bs = batch_ids.astype(jnp.int32)
    sorted32, inv32, boff16 = _build_sort_kernel()(xs, ys, zs, bs)
    sorted_idxs = sorted32.astype(jnp.int64)
    inv_sorted_idxs = inv32.astype(jnp.int64)
    batch_offsets = boff16[:9]
    max_counts = jnp.max(boff16[1:9] - boff16[:8]).astype(jnp.int64)
    return (features, sorted_idxs, inv_sorted_idxs, batch_offsets, max_counts)
